# baseline (speedup 1.0000x reference)
import functools

import jax
import jax.numpy as jnp
from jax import lax
from jax.experimental import pallas as pl
from jax.experimental.pallas import tpu as pltpu
from jax.experimental.pallas import tpu_sc as plsc

VOCAB = 1000000
SEQ = 200
DIM = 64
BATCH = 4096

NC = 2
NS = 16
NW = NC * NS
LANES = 16

ROWS = BATCH * SEQ
RPW = ROWS // NW
SUB = 100
SEQS_PER_CHUNK = 2
CHUNK = SEQ * SEQS_PER_CHUNK
SUBS_PER_CHUNK = CHUNK // SUB
NCHUNK = RPW // CHUNK
NSUB = RPW // SUB
SCALE = 8.0

_mesh = plsc.VectorSubcoreMesh(core_axis_name="c", subcore_axis_name="s")


@functools.partial(
    pl.kernel,
    out_type=jax.ShapeDtypeStruct((ROWS, DIM), jnp.float32),
    mesh=_mesh,
    compiler_params=pltpu.CompilerParams(use_tc_tiling_on_sc=False),
    scratch_types=[
        pltpu.VMEM((NSUB, SUB), jnp.int32),
        pltpu.VMEM((CHUNK, DIM), jnp.float32),
        pltpu.VMEM((SEQ, DIM), jnp.float32),
        pltpu.SemaphoreType.DMA,
    ],
)
def _embed(idx_hbm, tok_hbm, pos_hbm, out_hbm, idx_v, rows_v, pos_v, sem):
    wid = lax.axis_index("s") * NC + lax.axis_index("c")
    pltpu.sync_copy(idx_hbm.at[wid], idx_v)
    pltpu.sync_copy(pos_hbm, pos_v)
    base = wid * RPW

    def chunk_body(c, carry):
        for k in range(SUBS_PER_CHUNK):
            pltpu.async_copy(
                tok_hbm.at[idx_v.at[c * SUBS_PER_CHUNK + k]],
                rows_v.at[pl.ds(k * SUB, SUB)],
                sem,
            ).wait()

        def row_body(r, carry2):
            for g in range(DIM // LANES):
                sl = pl.ds(g * LANES, LANES)
                p = pos_v[r, sl]
                for s in range(SEQS_PER_CHUNK):
                    row = s * SEQ + r
                    rows_v[row, sl] = rows_v[row, sl] * SCALE + p
            return carry2

        lax.fori_loop(0, SEQ, row_body, 0)
        pltpu.sync_copy(rows_v, out_hbm.at[pl.ds(base + c * CHUNK, CHUNK)])
        return carry

    lax.fori_loop(0, NCHUNK, chunk_body, 0)


def kernel(inputs, token_table, position_table):
    idx = inputs.reshape(NW, NSUB, SUB)
    out = _embed(idx, token_table, position_table)
    return out.reshape(BATCH, SEQ, DIM)
